# baseline (speedup 1.0000x reference)
import functools
import math

import jax
import jax.numpy as jnp
import numpy as np
from jax import lax
from jax.experimental import pallas as pl
from jax.experimental.pallas import tpu as pltpu
from jax.experimental.pallas import tpu_sc as plsc

D_MODEL = 128
CONTEXT = 2048
B, S = 32, 2048

NC, NS = 2, 16
NW = NC * NS
GB = 16
PW = 128
N_BGROUP = B // GB
N_PBAND = S // PW
GATHER_ROWS = 128
CHUNK_ROWS = 128
N_GATHERS = (GB * PW) // GATHER_ROWS
N_CHUNKS = (GB * PW) // CHUNK_ROWS
NBUF = 6
LOOKAHEAD = NBUF - 2
NLANE = 16
NCOL = D_MODEL // NLANE


def _make_pe_consts():
    div_term = np.exp(
        np.arange(0, D_MODEL, 2, dtype=np.float64) * (-math.log(10000.0) / D_MODEL)
    )
    lane_freq = (8 * np.arange(NCOL)[:, None] + np.arange(NLANE)[None, :] // 2)
    w = div_term[lane_freq]
    pc = np.zeros((2 * N_PBAND + 2, 128), dtype=np.float32)
    for pb in range(N_PBAND):
        p0 = pb * PW
        for j in range(NCOL):
            col = (j % 4) * 32
            pc[2 * pb + j // 4, col:col + 16] = 2.0 * np.sin(p0 * w[j])
            pc[2 * pb + j // 4, col + 16:col + 32] = 2.0 * np.cos(p0 * w[j])
    for j in range(NCOL):
        col = (j % 4) * 32
        pc[2 * N_PBAND + j // 4, col:col + 16] = np.cos(w[j])
        pc[2 * N_PBAND + j // 4, col + 16:col + 32] = np.sin(w[j])
    return pc


_PE_CONSTS = _make_pe_consts()

_mesh = plsc.VectorSubcoreMesh(core_axis_name="c", subcore_axis_name="s")


@functools.partial(
    pl.kernel,
    mesh=_mesh,
    compiler_params=pltpu.CompilerParams(use_tc_tiling_on_sc=True),
    out_type=jax.ShapeDtypeStruct((B, S, D_MODEL), jnp.float32),
    scratch_types=[
        pltpu.VMEM((N_GATHERS, GATHER_ROWS), jnp.int32),
        pltpu.VMEM((PW, D_MODEL), jnp.float32),
        pltpu.VMEM((4, 128), jnp.float32),
        pltpu.VMEM((NBUF, CHUNK_ROWS, D_MODEL), jnp.float32),
        pltpu.SemaphoreType.DMA,
        pltpu.SemaphoreType.DMA,
        pltpu.SemaphoreType.DMA,
        pltpu.SemaphoreType.DMA,
    ],
)
def _embed(table_hbm, x_hbm, pc_hbm, out_hbm, idx_v, pe_v, const_v, rows_v,
           sem_idx, sem_pe, sem_g, sem_w):
    wid = lax.axis_index("s") * NC + lax.axis_index("c")
    bg = wid // N_PBAND
    pb = wid % N_PBAND
    b0 = bg * GB
    p0 = pb * PW

    idx_descs = [
        pltpu.async_copy(
            x_hbm.at[2 * bg + r // 8, pb, r % 8, :],
            idx_v.at[r],
            sem_idx,
        )
        for r in range(N_GATHERS)
    ]
    base_desc = pltpu.async_copy(
        pc_hbm.at[pl.ds(2 * pb, 2), :], const_v.at[pl.ds(0, 2), :], sem_pe
    )
    rot_desc = pltpu.async_copy(
        pc_hbm.at[pl.ds(2 * N_PBAND, 2), :], const_v.at[pl.ds(2, 2), :], sem_pe
    )
    for d in idx_descs:
        d.wait()

    def fire_gather(c):
        return pltpu.async_copy(
            table_hbm.at[idx_v.at[c]],
            rows_v.at[c % NBUF],
            sem_g,
        )

    def fire_write(c):
        return pltpu.async_copy(
            rows_v.at[c % NBUF],
            out_hbm.at[b0 + c, pl.ds(p0, PW), :],
            sem_w,
        )

    def compute(c):
        buf = c % NBUF

        @plsc.parallel_loop(0, PW, unroll=2)
        def _(i):
            for j in range(NCOL):
                sl = pl.ds(j * NLANE, NLANE)
                p = pe_v[i, sl]
                e = rows_v[buf, i, sl]
                rows_v[buf, i, sl] = e + e + p

    g_descs = {c: fire_gather(c) for c in range(LOOKAHEAD)}
    w_descs = {}

    base_desc.wait()
    rot_desc.wait()
    par = (lax.iota(jnp.int32, NLANE) & 1).astype(jnp.float32)
    inv = 1.0 - par
    for j in range(NCOL):
        col = (j % 4) * 32
        cw = const_v[2 + j // 4, pl.ds(col, NLANE)]
        sw = const_v[2 + j // 4, pl.ds(col + 16, NLANE)]

        def pe_gen(p, carry, j=j, cw=cw, sw=sw):
            ss, cc = carry
            pe_v[p, pl.ds(j * NLANE, NLANE)] = ss * inv + cc * par
            return ss * cw + cc * sw, cc * cw - ss * sw

        lax.fori_loop(
            0, PW, pe_gen,
            (
                const_v[j // 4, pl.ds(col, NLANE)],
                const_v[j // 4, pl.ds(col + 16, NLANE)],
            ),
        )

    for c in range(N_CHUNKS):
        g_descs.pop(c).wait()
        compute(c)
        w_descs[c] = fire_write(c)
        if c + LOOKAHEAD < N_CHUNKS:
            prev = c + LOOKAHEAD - NBUF
            if prev >= 0:
                w_descs.pop(prev).wait()
            g_descs[c + LOOKAHEAD] = fire_gather(c + LOOKAHEAD)
    for d in w_descs.values():
        d.wait()


def kernel(x, table):
    x4 = x.astype(jnp.int32).reshape(4, 8, 16, 128).transpose(0, 2, 1, 3)
    return _embed(table, x4, jnp.asarray(_PE_CONSTS))


# speedup vs baseline: 1.0526x; 1.0526x over previous
import functools
import math

import jax
import jax.numpy as jnp
import numpy as np
from jax import lax
from jax.experimental import pallas as pl
from jax.experimental.pallas import tpu as pltpu
from jax.experimental.pallas import tpu_sc as plsc

D_MODEL = 128
CONTEXT = 2048
B, S = 32, 2048

NC, NS = 2, 16
NW = NC * NS
GB = 16
PW = 128
N_BGROUP = B // GB
N_PBAND = S // PW
GATHER_ROWS = 128
CHUNK_ROWS = 128
N_GATHERS = (GB * PW) // GATHER_ROWS
N_CHUNKS = (GB * PW) // CHUNK_ROWS
NBUF = 6
LOOKAHEAD = NBUF - 2
NLANE = 16
NCOL = D_MODEL // NLANE


def _make_pe_consts():
    div_term = np.exp(
        np.arange(0, D_MODEL, 2, dtype=np.float64) * (-math.log(10000.0) / D_MODEL)
    )
    lane_freq = (8 * np.arange(NCOL)[:, None] + np.arange(NLANE)[None, :] // 2)
    w = div_term[lane_freq]
    pc = np.zeros((2 * N_PBAND + 2, 128), dtype=np.float32)
    for pb in range(N_PBAND):
        p0 = pb * PW
        for j in range(NCOL):
            col = (j % 4) * 32
            pc[2 * pb + j // 4, col:col + 16] = 2.0 * np.sin(p0 * w[j])
            pc[2 * pb + j // 4, col + 16:col + 32] = 2.0 * np.cos(p0 * w[j])
    for j in range(NCOL):
        col = (j % 4) * 32
        pc[2 * N_PBAND + j // 4, col:col + 16] = np.cos(w[j])
        pc[2 * N_PBAND + j // 4, col + 16:col + 32] = np.sin(w[j])
    return pc


_PE_CONSTS = _make_pe_consts()

_mesh = plsc.VectorSubcoreMesh(core_axis_name="c", subcore_axis_name="s")


@functools.partial(
    pl.kernel,
    mesh=_mesh,
    compiler_params=pltpu.CompilerParams(use_tc_tiling_on_sc=True),
    out_type=jax.ShapeDtypeStruct((B, S, D_MODEL), jnp.float32),
    scratch_types=[
        pltpu.VMEM((N_GATHERS, GATHER_ROWS), jnp.int32),
        pltpu.VMEM((PW, D_MODEL), jnp.float32),
        pltpu.VMEM((4, 128), jnp.float32),
        pltpu.VMEM((NBUF, CHUNK_ROWS, D_MODEL), jnp.float32),
        pltpu.SemaphoreType.DMA,
        pltpu.SemaphoreType.DMA,
        pltpu.SemaphoreType.DMA,
        pltpu.SemaphoreType.DMA,
    ],
)
def _embed(table_hbm, x_hbm, pc_hbm, out_hbm, idx_v, pe_v, const_v, rows_v,
           sem_idx, sem_pe, sem_g, sem_w):
    wid = lax.axis_index("s") * NC + lax.axis_index("c")
    bg = wid // N_PBAND
    pb = wid % N_PBAND
    b0 = bg * GB
    p0 = pb * PW

    idx_descs = [
        pltpu.async_copy(
            x_hbm.at[2 * bg + r // 8, pb, r % 8, :],
            idx_v.at[r],
            sem_idx,
        )
        for r in range(N_GATHERS)
    ]
    base_desc = pltpu.async_copy(
        pc_hbm.at[pl.ds(2 * pb, 2), :], const_v.at[pl.ds(0, 2), :], sem_pe
    )
    rot_desc = pltpu.async_copy(
        pc_hbm.at[pl.ds(2 * N_PBAND, 2), :], const_v.at[pl.ds(2, 2), :], sem_pe
    )
    for d in idx_descs:
        d.wait()

    def fire_gather(c):
        return pltpu.async_copy(
            table_hbm.at[idx_v.at[c]],
            rows_v.at[c % NBUF],
            sem_g,
        )

    def fire_write(c):
        return pltpu.async_copy(
            rows_v.at[c % NBUF],
            out_hbm.at[b0 + c, pl.ds(p0, PW), :],
            sem_w,
        )

    def compute(c):
        buf = c % NBUF

        @plsc.parallel_loop(0, PW, unroll=2)
        def _(i):
            for j in range(NCOL):
                sl = pl.ds(j * NLANE, NLANE)
                p = pe_v[i, sl]
                e = rows_v[buf, i, sl]
                rows_v[buf, i, sl] = e + e + p

    g_descs = {c: fire_gather(c) for c in range(LOOKAHEAD)}
    w_descs = {}

    base_desc.wait()
    rot_desc.wait()
    par = (lax.iota(jnp.int32, NLANE) & 1).astype(jnp.float32)
    inv = 1.0 - par
    cws = [const_v[2 + j // 4, pl.ds((j % 4) * 32, NLANE)] for j in range(NCOL)]
    sws = [const_v[2 + j // 4, pl.ds((j % 4) * 32 + 16, NLANE)]
           for j in range(NCOL)]

    def pe_gen(p, carry):
        ss, cc = carry
        new_ss, new_cc = [], []
        for j in range(NCOL):
            pe_v[p, pl.ds(j * NLANE, NLANE)] = ss[j] * inv + cc[j] * par
            new_ss.append(ss[j] * cws[j] + cc[j] * sws[j])
            new_cc.append(cc[j] * cws[j] - ss[j] * sws[j])
        return tuple(new_ss), tuple(new_cc)

    lax.fori_loop(
        0, PW, pe_gen,
        (
            tuple(const_v[j // 4, pl.ds((j % 4) * 32, NLANE)]
                  for j in range(NCOL)),
            tuple(const_v[j // 4, pl.ds((j % 4) * 32 + 16, NLANE)]
                  for j in range(NCOL)),
        ),
    )

    for c in range(N_CHUNKS):
        g_descs.pop(c).wait()
        compute(c)
        w_descs[c] = fire_write(c)
        if c + LOOKAHEAD < N_CHUNKS:
            prev = c + LOOKAHEAD - NBUF
            if prev >= 0:
                w_descs.pop(prev).wait()
            g_descs[c + LOOKAHEAD] = fire_gather(c + LOOKAHEAD)
    for d in w_descs.values():
        d.wait()


def kernel(x, table):
    x4 = x.astype(jnp.int32).reshape(4, 8, 16, 128).transpose(0, 2, 1, 3)
    return _embed(table, x4, jnp.asarray(_PE_CONSTS))


# speedup vs baseline: 1.0637x; 1.0105x over previous
import functools
import math

import jax
import jax.numpy as jnp
import numpy as np
from jax import lax
from jax.experimental import pallas as pl
from jax.experimental.pallas import tpu as pltpu
from jax.experimental.pallas import tpu_sc as plsc

D_MODEL = 128
CONTEXT = 2048
B, S = 32, 2048

NC, NS = 2, 16
NW = NC * NS
GB = 16
PW = 128
N_BGROUP = B // GB
N_PBAND = S // PW
GATHER_ROWS = 128
CHUNK_ROWS = 128
N_GATHERS = (GB * PW) // GATHER_ROWS
N_CHUNKS = (GB * PW) // CHUNK_ROWS
NBUF = 6
LOOKAHEAD = NBUF - 2
NLANE = 16
NCOL = D_MODEL // NLANE


def _make_pe2():
    position = np.arange(CONTEXT, dtype=np.float32)[:, None]
    div_term = np.exp(
        np.arange(0, D_MODEL, 2, dtype=np.float32) * (-math.log(10000.0) / D_MODEL)
    )
    pe = np.zeros((CONTEXT, D_MODEL), dtype=np.float32)
    pe[:, 0::2] = np.sin(position * div_term)
    pe[:, 1::2] = np.cos(position * div_term)
    return 2.0 * pe


_PE2 = _make_pe2()

_mesh = plsc.VectorSubcoreMesh(core_axis_name="c", subcore_axis_name="s")


@functools.partial(
    pl.kernel,
    mesh=_mesh,
    compiler_params=pltpu.CompilerParams(use_tc_tiling_on_sc=True),
    out_type=jax.ShapeDtypeStruct((B, S, D_MODEL), jnp.float32),
    scratch_types=[
        pltpu.VMEM((N_GATHERS, GATHER_ROWS), jnp.int32),
        pltpu.VMEM((PW, D_MODEL), jnp.float32),
        pltpu.VMEM((NBUF, CHUNK_ROWS, D_MODEL), jnp.float32),
        pltpu.SemaphoreType.DMA,
        pltpu.SemaphoreType.DMA,
        pltpu.SemaphoreType.DMA,
        pltpu.SemaphoreType.DMA,
    ],
)
def _embed(table_hbm, x_hbm, pe2_hbm, out_hbm, idx_v, pe_v, rows_v,
           sem_idx, sem_pe, sem_g, sem_w):
    wid = lax.axis_index("s") * NC + lax.axis_index("c")
    bg = wid // N_PBAND
    pb = wid % N_PBAND
    b0 = bg * GB
    p0 = pb * PW

    idx_descs = [
        pltpu.async_copy(
            x_hbm.at[2 * bg + r // 8, pb, r % 8, :],
            idx_v.at[r],
            sem_idx,
        )
        for r in range(N_GATHERS)
    ]
    pe_desc = pltpu.async_copy(pe2_hbm.at[pl.ds(p0, PW), :], pe_v, sem_pe)
    for d in idx_descs:
        d.wait()

    def fire_gather(c):
        return pltpu.async_copy(
            table_hbm.at[idx_v.at[c]],
            rows_v.at[c % NBUF],
            sem_g,
        )

    def fire_write(c):
        return pltpu.async_copy(
            rows_v.at[c % NBUF],
            out_hbm.at[b0 + c, pl.ds(p0, PW), :],
            sem_w,
        )

    def compute(c):
        buf = c % NBUF

        @plsc.parallel_loop(0, PW, unroll=2)
        def _(i):
            for j in range(NCOL):
                sl = pl.ds(j * NLANE, NLANE)
                p = pe_v[i, sl]
                e = rows_v[buf, i, sl]
                rows_v[buf, i, sl] = e + e + p

    g_descs = {c: fire_gather(c) for c in range(LOOKAHEAD)}
    w_descs = {}
    pe_desc.wait()
    for c in range(N_CHUNKS):
        g_descs.pop(c).wait()
        compute(c)
        w_descs[c] = fire_write(c)
        if c + LOOKAHEAD < N_CHUNKS:
            prev = c + LOOKAHEAD - NBUF
            if prev >= 0:
                w_descs.pop(prev).wait()
            g_descs[c + LOOKAHEAD] = fire_gather(c + LOOKAHEAD)
    for d in w_descs.values():
        d.wait()


def kernel(x, table):
    x4 = x.astype(jnp.int32).reshape(4, 8, 16, 128).transpose(0, 2, 1, 3)
    return _embed(table, x4, jnp.asarray(_PE2))
